# baseline (speedup 1.0000x reference)
import functools

import jax
import jax.numpy as jnp
from jax import lax
from jax.experimental import pallas as pl
from jax.experimental.pallas import tpu as pltpu

B, H, W = 32, 320, 320
HW = H * W
ROWS = HW // 128
N = B * HW
HNM_RATIO = 5
K_ELSE = int(N * 0.1)
MASK30 = (1 << 30) - 1

_RA = 80
_RH = 80

def _srl(x, n):
    return lax.shift_right_logical(x, jnp.int32(n))


def _pass_a(ps_ref, gl_ref, gm2_ref, gm3_ref, gm4_ref, gm5_ref, pbb_ref,
            enc_ref, w_ref, stats_ref, hist_ref):
    b = pl.program_id(0)
    i = pl.program_id(1)
    first = jnp.logical_and(b == 0, i == 0)

    s0 = ps_ref[0, 0, :, :]
    s1 = ps_ref[0, 1, :, :]
    g0 = gl_ref[0, 0, :, :]
    g1 = gl_ref[0, 1, :, :]

    m = jnp.maximum(s0, s1)
    e0 = jnp.exp(s0 - m)
    e1 = jnp.exp(s1 - m)
    z = e0 + e1
    p1 = e1 / z
    logz = jnp.log(z)
    logp0 = (s0 - m) - logz
    logp1 = (s1 - m) - logz

    posf = g0 > 0.5
    negf = g1 > 0.5

    bits = pltpu.bitcast(p1, jnp.int32)
    wv = -g1 * logp1
    enc_ref[0, :, :] = bits | jnp.where(negf, jnp.int32(1 << 30), jnp.int32(0))
    w_ref[0, :, :] = wv

    pos_cnt = jnp.sum(jnp.where(posf, 1.0, 0.0))
    neg_cnt = jnp.sum(jnp.where(negf, 1.0, 0.0))
    term0 = jnp.sum(jnp.where(posf, -g0 * logp0, 0.0))

    sq = jnp.float32(0.0)
    ms = jnp.float32(0.0)
    gm_refs = (gm2_ref, gm3_ref, gm4_ref, gm5_ref)
    for c in range(4):
        mm = gm_refs[c][0, 0, :, :]
        d = pbb_ref[0, c, :, :] * mm - gl_ref[0, 2 + c, :, :] * mm
        sq = sq + jnp.sum(d * d)
        ms = ms + jnp.sum(mm)

    e = _RA * 128
    x = _srl(bits.reshape(1, e), 20)
    xneg = jnp.where(jnp.where(negf, 1, 0).reshape(1, e) == 1, x, 2047)
    xh = _srl(x, 5).astype(jnp.int16)
    xnh = _srl(xneg, 5).astype(jnp.int16)
    xl = (x & 31).astype(jnp.int16)
    iota = lax.broadcasted_iota(jnp.int16, (32, 1), 0)
    one = jnp.bfloat16(1.0)
    zero = jnp.bfloat16(0.0)
    oh_all = jnp.where(xh == iota, one, zero)
    oh_neg = jnp.where(xnh == iota, one, zero)
    lhs = jnp.concatenate([oh_all, oh_neg], axis=0)
    oh_lo = jnp.where(xl == iota, one, zero)
    h = lax.dot_general(lhs, oh_lo, (((1,), (1,)), ((), ())),
                        preferred_element_type=jnp.float32)

    row = lax.broadcasted_iota(jnp.int32, (8, 128), 0)
    col = lax.broadcasted_iota(jnp.int32, (8, 128), 1)
    r0 = row == 0
    sv = (jnp.where(r0 & (col == 0), pos_cnt, 0.0)
          + jnp.where(r0 & (col == 1), neg_cnt, 0.0)
          + jnp.where(r0 & (col == 2), term0, 0.0)
          + jnp.where(r0 & (col == 3), sq, 0.0)
          + jnp.where(r0 & (col == 4), ms, 0.0))

    @pl.when(first)
    def _():
        stats_ref[:, :] = sv
        hist_ref[:, :] = h

    @pl.when(jnp.logical_not(first))
    def _():
        stats_ref[:, :] += sv
        hist_ref[:, :] += h


def _pass_hist(shift, params_ref, enc_ref, hist_ref):
    b = pl.program_id(0)
    i = pl.program_id(1)
    first = jnp.logical_and(b == 0, i == 0)

    has_pos = params_ref[0]
    sel_hi = params_ref[1]

    e = _RH * 128
    enc = enc_ref[0, :, :].reshape(1, e)
    neg = _srl(enc, 30)
    pb = enc & MASK30
    xb = jnp.where(has_pos == 1, jnp.where(neg == 1, pb, 0), pb)

    match = _srl(xb, shift + 10) == sel_hi
    x = jnp.where(match, _srl(xb, shift) & 1023, 2047)

    xh = _srl(x, 5).astype(jnp.int16)
    xl = (x & 31).astype(jnp.int16)
    iota = lax.broadcasted_iota(jnp.int16, (32, 1), 0)
    one = jnp.bfloat16(1.0)
    zero = jnp.bfloat16(0.0)
    oh_hi = jnp.where(xh == iota, one, zero)
    oh_lo = jnp.where(xl == iota, one, zero)
    h = lax.dot_general(oh_hi, oh_lo, (((1,), (1,)), ((), ())),
                        preferred_element_type=jnp.float32)

    @pl.when(first)
    def _():
        hist_ref[:, :] = h

    @pl.when(jnp.logical_not(first))
    def _():
        hist_ref[:, :] += h


def _pass_l3(params_ref, enc_ref, w_ref, hist_ref, tstats_ref):
    b = pl.program_id(0)
    i = pl.program_id(1)
    first = jnp.logical_and(b == 0, i == 0)

    has_pos = params_ref[0]
    pref2 = params_ref[1]

    e = _RH * 128
    enc = enc_ref[0, :, :].reshape(1, e)
    wr = w_ref[0, :, :].reshape(1, e)
    neg = _srl(enc, 30)
    pb = enc & MASK30
    xb = jnp.where(has_pos == 1, jnp.where(neg == 1, pb, 0), pb)

    t_below = jnp.sum(jnp.where(xb < pref2 * 1024, wr, 0.0))

    match = _srl(xb, 10) == pref2
    x = jnp.where(match, xb & 1023, 2047)
    xh = _srl(x, 5).astype(jnp.int16)
    xl = (x & 31).astype(jnp.int16)
    iota = lax.broadcasted_iota(jnp.int16, (32, 1), 0)
    one = jnp.bfloat16(1.0)
    zero = jnp.bfloat16(0.0)
    oh_hi = jnp.where(xh == iota, one, zero)
    oh_lo = jnp.where(xl == iota, one, zero)
    wb = wr.astype(jnp.bfloat16)
    lhs = jnp.concatenate([oh_hi, oh_hi * wb], axis=0)
    h = lax.dot_general(lhs, oh_lo, (((1,), (1,)), ((), ())),
                        preferred_element_type=jnp.float32)

    row = lax.broadcasted_iota(jnp.int32, (8, 128), 0)
    col = lax.broadcasted_iota(jnp.int32, (8, 128), 1)
    sv = jnp.where((row == 0) & (col == 0), t_below, 0.0)

    @pl.when(first)
    def _():
        hist_ref[:, :] = h
        tstats_ref[:, :] = sv

    @pl.when(jnp.logical_not(first))
    def _():
        hist_ref[:, :] += h
        tstats_ref[:, :] += sv


def _hist_call(enc, params, shift):
    return pl.pallas_call(
        functools.partial(_pass_hist, shift),
        grid=(B, ROWS // _RH),
        in_specs=[
            pl.BlockSpec(memory_space=pltpu.SMEM),
            pl.BlockSpec((1, _RH, 128), lambda b, i: (b, i, 0)),
        ],
        out_specs=pl.BlockSpec((32, 32), lambda b, i: (0, 0)),
        out_shape=jax.ShapeDtypeStruct((32, 32), jnp.float32),
    )(params, enc)


def kernel(pred_score, pred_bbox, gt_mask, gt_label):
    ps = pred_score.reshape(B, 2, ROWS, 128)
    pbb = pred_bbox.reshape(B, 4, ROWS, 128)
    gm = gt_mask.reshape(B, 6, ROWS, 128)
    gl = gt_label.reshape(B, 6, ROWS, 128)

    enc, w, stats, hist1 = pl.pallas_call(
        _pass_a,
        grid=(B, ROWS // _RA),
        in_specs=[
            pl.BlockSpec((1, 2, _RA, 128), lambda b, i: (b, 0, i, 0)),
            pl.BlockSpec((1, 6, _RA, 128), lambda b, i: (b, 0, i, 0)),
            pl.BlockSpec((1, 1, _RA, 128), lambda b, i: (b, 2, i, 0)),
            pl.BlockSpec((1, 1, _RA, 128), lambda b, i: (b, 3, i, 0)),
            pl.BlockSpec((1, 1, _RA, 128), lambda b, i: (b, 4, i, 0)),
            pl.BlockSpec((1, 1, _RA, 128), lambda b, i: (b, 5, i, 0)),
            pl.BlockSpec((1, 4, _RA, 128), lambda b, i: (b, 0, i, 0)),
        ],
        out_specs=[
            pl.BlockSpec((1, _RA, 128), lambda b, i: (b, i, 0)),
            pl.BlockSpec((1, _RA, 128), lambda b, i: (b, i, 0)),
            pl.BlockSpec((8, 128), lambda b, i: (0, 0)),
            pl.BlockSpec((64, 32), lambda b, i: (0, 0)),
        ],
        out_shape=[
            jax.ShapeDtypeStruct((B, ROWS, 128), jnp.int32),
            jax.ShapeDtypeStruct((B, ROWS, 128), jnp.float32),
            jax.ShapeDtypeStruct((8, 128), jnp.float32),
            jax.ShapeDtypeStruct((64, 32), jnp.float32),
        ],
    )(ps, gl, gm, gm, gm, gm, pbb)

    pos_num = stats[0, 0].astype(jnp.int32)
    neg_num = stats[0, 1].astype(jnp.int32)
    term0 = stats[0, 2]
    sq = stats[0, 3]
    mbs = stats[0, 4]
    has_pos = pos_num > 0
    hp = has_pos.astype(jnp.int32)

    nns = jnp.minimum(HNM_RATIO * pos_num, neg_num)
    k_pos = jnp.where(nns >= 1, nns, N)
    k = jnp.where(has_pos, k_pos, K_ELSE)

    cnt_all = hist1[0:32, :].reshape(1024).astype(jnp.int32)
    cnt_neg = hist1[32:64, :].reshape(1024).astype(jnp.int32)
    cnt_posb = cnt_neg.at[0].add(N - neg_num)
    h1 = jnp.where(has_pos, cnt_posb, cnt_all)

    c1 = jnp.cumsum(h1)
    sel1 = jnp.argmax(c1 >= k).astype(jnp.int32)
    e1 = c1[sel1] - h1[sel1]
    k2 = k - e1

    h2 = _hist_call(enc, jnp.stack([hp, sel1]), 10).reshape(1024).astype(jnp.int32)
    c2 = jnp.cumsum(h2)
    sel2 = jnp.argmax(c2 >= k2).astype(jnp.int32)
    e2 = c2[sel2] - h2[sel2]
    k3 = k2 - e2

    h3f, tstats = pl.pallas_call(
        _pass_l3,
        grid=(B, ROWS // _RH),
        in_specs=[
            pl.BlockSpec(memory_space=pltpu.SMEM),
            pl.BlockSpec((1, _RH, 128), lambda b, i: (b, i, 0)),
            pl.BlockSpec((1, _RH, 128), lambda b, i: (b, i, 0)),
        ],
        out_specs=[
            pl.BlockSpec((64, 32), lambda b, i: (0, 0)),
            pl.BlockSpec((8, 128), lambda b, i: (0, 0)),
        ],
        out_shape=[
            jax.ShapeDtypeStruct((64, 32), jnp.float32),
            jax.ShapeDtypeStruct((8, 128), jnp.float32),
        ],
    )(jnp.stack([hp, sel1 * 1024 + sel2]), enc, w)

    h3 = h3f[0:32, :].reshape(1024).astype(jnp.int32)
    w3 = h3f[32:64, :].reshape(1024)
    c3 = jnp.cumsum(h3)
    sel3 = jnp.argmax(c3 >= k3).astype(jnp.int32)
    i3 = c3[sel3]
    i3w = jnp.cumsum(w3)[sel3]

    n_sel = (pos_num + e1 + e2 + i3).astype(jnp.float32)
    term1 = tstats[0, 0] + i3w

    loss_score = (term0 + term1) / n_sel
    loss_bbox = jnp.where(mbs == 0, jnp.zeros_like(loss_score), sq / mbs)
    loss = loss_score + loss_bbox
    return (loss, loss_score, loss_bbox)


# speedup vs baseline: 1.3752x; 1.3752x over previous
import functools

import jax
import jax.numpy as jnp
from jax import lax
from jax.experimental import pallas as pl
from jax.experimental.pallas import tpu as pltpu

B, H, W = 32, 320, 320
HW = H * W
ROWS = HW // 128
N = B * HW
HNM_RATIO = 5
K_ELSE = int(N * 0.1)
MASK30 = (1 << 30) - 1

_RA = 200
_RH = 200

def _srl(x, n):
    return lax.shift_right_logical(x, jnp.int32(n))


def _pass_a(ps_ref, gl_ref, gm2_ref, gm3_ref, gm4_ref, gm5_ref, pbb_ref,
            enc_ref, w_ref, stats_ref, hist_ref):
    b = pl.program_id(0)
    i = pl.program_id(1)
    first = jnp.logical_and(b == 0, i == 0)

    s0 = ps_ref[0, 0, :, :]
    s1 = ps_ref[0, 1, :, :]
    g0 = gl_ref[0, 0, :, :]
    g1 = gl_ref[0, 1, :, :]

    m = jnp.maximum(s0, s1)
    e0 = jnp.exp(s0 - m)
    e1 = jnp.exp(s1 - m)
    z = e0 + e1
    p1 = e1 / z
    logz = jnp.log(z)
    logp0 = (s0 - m) - logz
    logp1 = (s1 - m) - logz

    posf = g0 > 0.5
    negf = g1 > 0.5

    bits = pltpu.bitcast(p1, jnp.int32)
    wv = -g1 * logp1
    enc_ref[0, :, :] = bits | jnp.where(negf, jnp.int32(1 << 30), jnp.int32(0))
    w_ref[0, :, :] = wv

    pos_cnt = jnp.sum(jnp.where(posf, 1.0, 0.0))
    neg_cnt = jnp.sum(jnp.where(negf, 1.0, 0.0))
    term0 = jnp.sum(jnp.where(posf, -g0 * logp0, 0.0))

    sq = jnp.float32(0.0)
    ms = jnp.float32(0.0)
    gm_refs = (gm2_ref, gm3_ref, gm4_ref, gm5_ref)
    for c in range(4):
        mm = gm_refs[c][0, 0, :, :]
        d = pbb_ref[0, c, :, :] * mm - gl_ref[0, 2 + c, :, :] * mm
        sq = sq + jnp.sum(d * d)
        ms = ms + jnp.sum(mm)

    e = _RA * 128
    x = _srl(bits.reshape(1, e), 20)
    xneg = jnp.where(jnp.where(negf, 1, 0).reshape(1, e) == 1, x, 2047)
    xh = _srl(x, 5).astype(jnp.int16)
    xnh = _srl(xneg, 5).astype(jnp.int16)
    xl = (x & 31).astype(jnp.int16)
    iota = lax.broadcasted_iota(jnp.int16, (32, 1), 0)
    one = jnp.bfloat16(1.0)
    zero = jnp.bfloat16(0.0)
    oh_all = jnp.where(xh == iota, one, zero)
    oh_neg = jnp.where(xnh == iota, one, zero)
    lhs = jnp.concatenate([oh_all, oh_neg], axis=0)
    oh_lo = jnp.where(xl == iota, one, zero)
    h = lax.dot_general(lhs, oh_lo, (((1,), (1,)), ((), ())),
                        preferred_element_type=jnp.float32)

    row = lax.broadcasted_iota(jnp.int32, (8, 128), 0)
    col = lax.broadcasted_iota(jnp.int32, (8, 128), 1)
    r0 = row == 0
    sv = (jnp.where(r0 & (col == 0), pos_cnt, 0.0)
          + jnp.where(r0 & (col == 1), neg_cnt, 0.0)
          + jnp.where(r0 & (col == 2), term0, 0.0)
          + jnp.where(r0 & (col == 3), sq, 0.0)
          + jnp.where(r0 & (col == 4), ms, 0.0))

    @pl.when(first)
    def _():
        stats_ref[:, :] = sv
        hist_ref[:, :] = h

    @pl.when(jnp.logical_not(first))
    def _():
        stats_ref[:, :] += sv
        hist_ref[:, :] += h


def _pass_hist(shift, params_ref, enc_ref, hist_ref):
    b = pl.program_id(0)
    i = pl.program_id(1)
    first = jnp.logical_and(b == 0, i == 0)

    has_pos = params_ref[0]
    sel_hi = params_ref[1]

    e = _RH * 128
    enc = enc_ref[0, :, :].reshape(1, e)
    neg = _srl(enc, 30)
    pb = enc & MASK30
    xb = jnp.where(has_pos == 1, jnp.where(neg == 1, pb, 0), pb)

    match = _srl(xb, shift + 10) == sel_hi
    x = jnp.where(match, _srl(xb, shift) & 1023, 2047)

    xh = _srl(x, 5).astype(jnp.int16)
    xl = (x & 31).astype(jnp.int16)
    iota = lax.broadcasted_iota(jnp.int16, (32, 1), 0)
    one = jnp.bfloat16(1.0)
    zero = jnp.bfloat16(0.0)
    oh_hi = jnp.where(xh == iota, one, zero)
    oh_lo = jnp.where(xl == iota, one, zero)
    h = lax.dot_general(oh_hi, oh_lo, (((1,), (1,)), ((), ())),
                        preferred_element_type=jnp.float32)

    @pl.when(first)
    def _():
        hist_ref[:, :] = h

    @pl.when(jnp.logical_not(first))
    def _():
        hist_ref[:, :] += h


def _pass_l3(params_ref, enc_ref, w_ref, hist_ref, tstats_ref):
    b = pl.program_id(0)
    i = pl.program_id(1)
    first = jnp.logical_and(b == 0, i == 0)

    has_pos = params_ref[0]
    pref2 = params_ref[1]

    e = _RH * 128
    enc = enc_ref[0, :, :].reshape(1, e)
    wr = w_ref[0, :, :].reshape(1, e)
    neg = _srl(enc, 30)
    pb = enc & MASK30
    xb = jnp.where(has_pos == 1, jnp.where(neg == 1, pb, 0), pb)

    t_below = jnp.sum(jnp.where(xb < pref2 * 1024, wr, 0.0))

    match = _srl(xb, 10) == pref2
    x = jnp.where(match, xb & 1023, 2047)
    xh = _srl(x, 5).astype(jnp.int16)
    xl = (x & 31).astype(jnp.int16)
    iota = lax.broadcasted_iota(jnp.int16, (32, 1), 0)
    one = jnp.bfloat16(1.0)
    zero = jnp.bfloat16(0.0)
    oh_hi = jnp.where(xh == iota, one, zero)
    oh_lo = jnp.where(xl == iota, one, zero)
    wb = wr.astype(jnp.bfloat16)
    lhs = jnp.concatenate([oh_hi, oh_hi * wb], axis=0)
    h = lax.dot_general(lhs, oh_lo, (((1,), (1,)), ((), ())),
                        preferred_element_type=jnp.float32)

    row = lax.broadcasted_iota(jnp.int32, (8, 128), 0)
    col = lax.broadcasted_iota(jnp.int32, (8, 128), 1)
    sv = jnp.where((row == 0) & (col == 0), t_below, 0.0)

    @pl.when(first)
    def _():
        hist_ref[:, :] = h
        tstats_ref[:, :] = sv

    @pl.when(jnp.logical_not(first))
    def _():
        hist_ref[:, :] += h
        tstats_ref[:, :] += sv


def _hist_call(enc, params, shift):
    return pl.pallas_call(
        functools.partial(_pass_hist, shift),
        grid=(B, ROWS // _RH),
        in_specs=[
            pl.BlockSpec(memory_space=pltpu.SMEM),
            pl.BlockSpec((1, _RH, 128), lambda b, i: (b, i, 0)),
        ],
        out_specs=pl.BlockSpec((32, 32), lambda b, i: (0, 0)),
        out_shape=jax.ShapeDtypeStruct((32, 32), jnp.float32),
    )(params, enc)


def kernel(pred_score, pred_bbox, gt_mask, gt_label):
    ps = pred_score.reshape(B, 2, ROWS, 128)
    pbb = pred_bbox.reshape(B, 4, ROWS, 128)
    gm = gt_mask.reshape(B, 6, ROWS, 128)
    gl = gt_label.reshape(B, 6, ROWS, 128)

    enc, w, stats, hist1 = pl.pallas_call(
        _pass_a,
        grid=(B, ROWS // _RA),
        in_specs=[
            pl.BlockSpec((1, 2, _RA, 128), lambda b, i: (b, 0, i, 0)),
            pl.BlockSpec((1, 6, _RA, 128), lambda b, i: (b, 0, i, 0)),
            pl.BlockSpec((1, 1, _RA, 128), lambda b, i: (b, 2, i, 0)),
            pl.BlockSpec((1, 1, _RA, 128), lambda b, i: (b, 3, i, 0)),
            pl.BlockSpec((1, 1, _RA, 128), lambda b, i: (b, 4, i, 0)),
            pl.BlockSpec((1, 1, _RA, 128), lambda b, i: (b, 5, i, 0)),
            pl.BlockSpec((1, 4, _RA, 128), lambda b, i: (b, 0, i, 0)),
        ],
        out_specs=[
            pl.BlockSpec((1, _RA, 128), lambda b, i: (b, i, 0)),
            pl.BlockSpec((1, _RA, 128), lambda b, i: (b, i, 0)),
            pl.BlockSpec((8, 128), lambda b, i: (0, 0)),
            pl.BlockSpec((64, 32), lambda b, i: (0, 0)),
        ],
        out_shape=[
            jax.ShapeDtypeStruct((B, ROWS, 128), jnp.int32),
            jax.ShapeDtypeStruct((B, ROWS, 128), jnp.float32),
            jax.ShapeDtypeStruct((8, 128), jnp.float32),
            jax.ShapeDtypeStruct((64, 32), jnp.float32),
        ],
    )(ps, gl, gm, gm, gm, gm, pbb)

    pos_num = stats[0, 0].astype(jnp.int32)
    neg_num = stats[0, 1].astype(jnp.int32)
    term0 = stats[0, 2]
    sq = stats[0, 3]
    mbs = stats[0, 4]
    has_pos = pos_num > 0
    hp = has_pos.astype(jnp.int32)

    nns = jnp.minimum(HNM_RATIO * pos_num, neg_num)
    k_pos = jnp.where(nns >= 1, nns, N)
    k = jnp.where(has_pos, k_pos, K_ELSE)

    cnt_all = hist1[0:32, :].reshape(1024).astype(jnp.int32)
    cnt_neg = hist1[32:64, :].reshape(1024).astype(jnp.int32)
    cnt_posb = cnt_neg.at[0].add(N - neg_num)
    h1 = jnp.where(has_pos, cnt_posb, cnt_all)

    c1 = jnp.cumsum(h1)
    sel1 = jnp.argmax(c1 >= k).astype(jnp.int32)
    e1 = c1[sel1] - h1[sel1]
    k2 = k - e1

    h2 = _hist_call(enc, jnp.stack([hp, sel1]), 10).reshape(1024).astype(jnp.int32)
    c2 = jnp.cumsum(h2)
    sel2 = jnp.argmax(c2 >= k2).astype(jnp.int32)
    e2 = c2[sel2] - h2[sel2]
    k3 = k2 - e2

    h3f, tstats = pl.pallas_call(
        _pass_l3,
        grid=(B, ROWS // _RH),
        in_specs=[
            pl.BlockSpec(memory_space=pltpu.SMEM),
            pl.BlockSpec((1, _RH, 128), lambda b, i: (b, i, 0)),
            pl.BlockSpec((1, _RH, 128), lambda b, i: (b, i, 0)),
        ],
        out_specs=[
            pl.BlockSpec((64, 32), lambda b, i: (0, 0)),
            pl.BlockSpec((8, 128), lambda b, i: (0, 0)),
        ],
        out_shape=[
            jax.ShapeDtypeStruct((64, 32), jnp.float32),
            jax.ShapeDtypeStruct((8, 128), jnp.float32),
        ],
    )(jnp.stack([hp, sel1 * 1024 + sel2]), enc, w)

    h3 = h3f[0:32, :].reshape(1024).astype(jnp.int32)
    w3 = h3f[32:64, :].reshape(1024)
    c3 = jnp.cumsum(h3)
    sel3 = jnp.argmax(c3 >= k3).astype(jnp.int32)
    i3 = c3[sel3]
    i3w = jnp.cumsum(w3)[sel3]

    n_sel = (pos_num + e1 + e2 + i3).astype(jnp.float32)
    term1 = tstats[0, 0] + i3w

    loss_score = (term0 + term1) / n_sel
    loss_bbox = jnp.where(mbs == 0, jnp.zeros_like(loss_score), sq / mbs)
    loss = loss_score + loss_bbox
    return (loss, loss_score, loss_bbox)


# speedup vs baseline: 1.4724x; 1.0707x over previous
import functools

import jax
import jax.numpy as jnp
from jax import lax
from jax.experimental import pallas as pl
from jax.experimental.pallas import tpu as pltpu

B, H, W = 32, 320, 320
HW = H * W
ROWS = HW // 128
N = B * HW
HNM_RATIO = 5
K_ELSE = int(N * 0.1)
MASK30 = (1 << 30) - 1

_RA = 400
_RH = 400

def _srl(x, n):
    return lax.shift_right_logical(x, jnp.int32(n))


def _pass_a(ps_ref, gl_ref, gm2_ref, gm3_ref, gm4_ref, gm5_ref, pbb_ref,
            enc_ref, w_ref, stats_ref, hist_ref):
    b = pl.program_id(0)
    i = pl.program_id(1)
    first = jnp.logical_and(b == 0, i == 0)

    s0 = ps_ref[0, 0, :, :]
    s1 = ps_ref[0, 1, :, :]
    g0 = gl_ref[0, 0, :, :]
    g1 = gl_ref[0, 1, :, :]

    m = jnp.maximum(s0, s1)
    e0 = jnp.exp(s0 - m)
    e1 = jnp.exp(s1 - m)
    z = e0 + e1
    p1 = e1 / z
    logz = jnp.log(z)
    logp0 = (s0 - m) - logz
    logp1 = (s1 - m) - logz

    posf = g0 > 0.5
    negf = g1 > 0.5

    bits = pltpu.bitcast(p1, jnp.int32)
    wv = -g1 * logp1
    enc_ref[0, :, :] = bits | jnp.where(negf, jnp.int32(1 << 30), jnp.int32(0))
    w_ref[0, :, :] = wv

    pos_cnt = jnp.sum(jnp.where(posf, 1.0, 0.0))
    neg_cnt = jnp.sum(jnp.where(negf, 1.0, 0.0))
    term0 = jnp.sum(jnp.where(posf, -g0 * logp0, 0.0))

    sq = jnp.float32(0.0)
    ms = jnp.float32(0.0)
    gm_refs = (gm2_ref, gm3_ref, gm4_ref, gm5_ref)
    for c in range(4):
        mm = gm_refs[c][0, 0, :, :]
        d = pbb_ref[0, c, :, :] * mm - gl_ref[0, 2 + c, :, :] * mm
        sq = sq + jnp.sum(d * d)
        ms = ms + jnp.sum(mm)

    e = _RA * 128
    x = _srl(bits.reshape(1, e), 20)
    xneg = jnp.where(jnp.where(negf, 1, 0).reshape(1, e) == 1, x, 2047)
    xh = _srl(x, 5).astype(jnp.int16)
    xnh = _srl(xneg, 5).astype(jnp.int16)
    xl = (x & 31).astype(jnp.int16)
    iota = lax.broadcasted_iota(jnp.int16, (32, 1), 0)
    one = jnp.bfloat16(1.0)
    zero = jnp.bfloat16(0.0)
    oh_all = jnp.where(xh == iota, one, zero)
    oh_neg = jnp.where(xnh == iota, one, zero)
    lhs = jnp.concatenate([oh_all, oh_neg], axis=0)
    oh_lo = jnp.where(xl == iota, one, zero)
    h = lax.dot_general(lhs, oh_lo, (((1,), (1,)), ((), ())),
                        preferred_element_type=jnp.float32)

    row = lax.broadcasted_iota(jnp.int32, (8, 128), 0)
    col = lax.broadcasted_iota(jnp.int32, (8, 128), 1)
    r0 = row == 0
    sv = (jnp.where(r0 & (col == 0), pos_cnt, 0.0)
          + jnp.where(r0 & (col == 1), neg_cnt, 0.0)
          + jnp.where(r0 & (col == 2), term0, 0.0)
          + jnp.where(r0 & (col == 3), sq, 0.0)
          + jnp.where(r0 & (col == 4), ms, 0.0))

    @pl.when(first)
    def _():
        stats_ref[:, :] = sv
        hist_ref[:, :] = h

    @pl.when(jnp.logical_not(first))
    def _():
        stats_ref[:, :] += sv
        hist_ref[:, :] += h


def _pass_hist(shift, params_ref, enc_ref, hist_ref):
    b = pl.program_id(0)
    i = pl.program_id(1)
    first = jnp.logical_and(b == 0, i == 0)

    has_pos = params_ref[0]
    sel_hi = params_ref[1]

    e = _RH * 128
    enc = enc_ref[0, :, :].reshape(1, e)
    neg = _srl(enc, 30)
    pb = enc & MASK30
    xb = jnp.where(has_pos == 1, jnp.where(neg == 1, pb, 0), pb)

    match = _srl(xb, shift + 10) == sel_hi
    x = jnp.where(match, _srl(xb, shift) & 1023, 2047)

    xh = _srl(x, 5).astype(jnp.int16)
    xl = (x & 31).astype(jnp.int16)
    iota = lax.broadcasted_iota(jnp.int16, (32, 1), 0)
    one = jnp.bfloat16(1.0)
    zero = jnp.bfloat16(0.0)
    oh_hi = jnp.where(xh == iota, one, zero)
    oh_lo = jnp.where(xl == iota, one, zero)
    h = lax.dot_general(oh_hi, oh_lo, (((1,), (1,)), ((), ())),
                        preferred_element_type=jnp.float32)

    @pl.when(first)
    def _():
        hist_ref[:, :] = h

    @pl.when(jnp.logical_not(first))
    def _():
        hist_ref[:, :] += h


def _pass_l3(params_ref, enc_ref, w_ref, hist_ref, tstats_ref):
    b = pl.program_id(0)
    i = pl.program_id(1)
    first = jnp.logical_and(b == 0, i == 0)

    has_pos = params_ref[0]
    pref2 = params_ref[1]

    e = _RH * 128
    enc = enc_ref[0, :, :].reshape(1, e)
    wr = w_ref[0, :, :].reshape(1, e)
    neg = _srl(enc, 30)
    pb = enc & MASK30
    xb = jnp.where(has_pos == 1, jnp.where(neg == 1, pb, 0), pb)

    t_below = jnp.sum(jnp.where(xb < pref2 * 1024, wr, 0.0))

    match = _srl(xb, 10) == pref2
    x = jnp.where(match, xb & 1023, 2047)
    xh = _srl(x, 5).astype(jnp.int16)
    xl = (x & 31).astype(jnp.int16)
    iota = lax.broadcasted_iota(jnp.int16, (32, 1), 0)
    one = jnp.bfloat16(1.0)
    zero = jnp.bfloat16(0.0)
    oh_hi = jnp.where(xh == iota, one, zero)
    oh_lo = jnp.where(xl == iota, one, zero)
    wb = wr.astype(jnp.bfloat16)
    lhs = jnp.concatenate([oh_hi, oh_hi * wb], axis=0)
    h = lax.dot_general(lhs, oh_lo, (((1,), (1,)), ((), ())),
                        preferred_element_type=jnp.float32)

    row = lax.broadcasted_iota(jnp.int32, (8, 128), 0)
    col = lax.broadcasted_iota(jnp.int32, (8, 128), 1)
    sv = jnp.where((row == 0) & (col == 0), t_below, 0.0)

    @pl.when(first)
    def _():
        hist_ref[:, :] = h
        tstats_ref[:, :] = sv

    @pl.when(jnp.logical_not(first))
    def _():
        hist_ref[:, :] += h
        tstats_ref[:, :] += sv


def _hist_call(enc, params, shift):
    return pl.pallas_call(
        functools.partial(_pass_hist, shift),
        grid=(B, ROWS // _RH),
        in_specs=[
            pl.BlockSpec(memory_space=pltpu.SMEM),
            pl.BlockSpec((1, _RH, 128), lambda b, i: (b, i, 0)),
        ],
        out_specs=pl.BlockSpec((32, 32), lambda b, i: (0, 0)),
        out_shape=jax.ShapeDtypeStruct((32, 32), jnp.float32),
    )(params, enc)


def kernel(pred_score, pred_bbox, gt_mask, gt_label):
    ps = pred_score.reshape(B, 2, ROWS, 128)
    pbb = pred_bbox.reshape(B, 4, ROWS, 128)
    gm = gt_mask.reshape(B, 6, ROWS, 128)
    gl = gt_label.reshape(B, 6, ROWS, 128)

    enc, w, stats, hist1 = pl.pallas_call(
        _pass_a,
        grid=(B, ROWS // _RA),
        in_specs=[
            pl.BlockSpec((1, 2, _RA, 128), lambda b, i: (b, 0, i, 0)),
            pl.BlockSpec((1, 6, _RA, 128), lambda b, i: (b, 0, i, 0)),
            pl.BlockSpec((1, 1, _RA, 128), lambda b, i: (b, 2, i, 0)),
            pl.BlockSpec((1, 1, _RA, 128), lambda b, i: (b, 3, i, 0)),
            pl.BlockSpec((1, 1, _RA, 128), lambda b, i: (b, 4, i, 0)),
            pl.BlockSpec((1, 1, _RA, 128), lambda b, i: (b, 5, i, 0)),
            pl.BlockSpec((1, 4, _RA, 128), lambda b, i: (b, 0, i, 0)),
        ],
        out_specs=[
            pl.BlockSpec((1, _RA, 128), lambda b, i: (b, i, 0)),
            pl.BlockSpec((1, _RA, 128), lambda b, i: (b, i, 0)),
            pl.BlockSpec((8, 128), lambda b, i: (0, 0)),
            pl.BlockSpec((64, 32), lambda b, i: (0, 0)),
        ],
        out_shape=[
            jax.ShapeDtypeStruct((B, ROWS, 128), jnp.int32),
            jax.ShapeDtypeStruct((B, ROWS, 128), jnp.float32),
            jax.ShapeDtypeStruct((8, 128), jnp.float32),
            jax.ShapeDtypeStruct((64, 32), jnp.float32),
        ],
    )(ps, gl, gm, gm, gm, gm, pbb)

    pos_num = stats[0, 0].astype(jnp.int32)
    neg_num = stats[0, 1].astype(jnp.int32)
    term0 = stats[0, 2]
    sq = stats[0, 3]
    mbs = stats[0, 4]
    has_pos = pos_num > 0
    hp = has_pos.astype(jnp.int32)

    nns = jnp.minimum(HNM_RATIO * pos_num, neg_num)
    k_pos = jnp.where(nns >= 1, nns, N)
    k = jnp.where(has_pos, k_pos, K_ELSE)

    cnt_all = hist1[0:32, :].reshape(1024).astype(jnp.int32)
    cnt_neg = hist1[32:64, :].reshape(1024).astype(jnp.int32)
    cnt_posb = cnt_neg.at[0].add(N - neg_num)
    h1 = jnp.where(has_pos, cnt_posb, cnt_all)

    c1 = jnp.cumsum(h1)
    sel1 = jnp.argmax(c1 >= k).astype(jnp.int32)
    e1 = c1[sel1] - h1[sel1]
    k2 = k - e1

    h2 = _hist_call(enc, jnp.stack([hp, sel1]), 10).reshape(1024).astype(jnp.int32)
    c2 = jnp.cumsum(h2)
    sel2 = jnp.argmax(c2 >= k2).astype(jnp.int32)
    e2 = c2[sel2] - h2[sel2]
    k3 = k2 - e2

    h3f, tstats = pl.pallas_call(
        _pass_l3,
        grid=(B, ROWS // _RH),
        in_specs=[
            pl.BlockSpec(memory_space=pltpu.SMEM),
            pl.BlockSpec((1, _RH, 128), lambda b, i: (b, i, 0)),
            pl.BlockSpec((1, _RH, 128), lambda b, i: (b, i, 0)),
        ],
        out_specs=[
            pl.BlockSpec((64, 32), lambda b, i: (0, 0)),
            pl.BlockSpec((8, 128), lambda b, i: (0, 0)),
        ],
        out_shape=[
            jax.ShapeDtypeStruct((64, 32), jnp.float32),
            jax.ShapeDtypeStruct((8, 128), jnp.float32),
        ],
    )(jnp.stack([hp, sel1 * 1024 + sel2]), enc, w)

    h3 = h3f[0:32, :].reshape(1024).astype(jnp.int32)
    w3 = h3f[32:64, :].reshape(1024)
    c3 = jnp.cumsum(h3)
    sel3 = jnp.argmax(c3 >= k3).astype(jnp.int32)
    i3 = c3[sel3]
    i3w = jnp.cumsum(w3)[sel3]

    n_sel = (pos_num + e1 + e2 + i3).astype(jnp.float32)
    term1 = tstats[0, 0] + i3w

    loss_score = (term0 + term1) / n_sel
    loss_bbox = jnp.where(mbs == 0, jnp.zeros_like(loss_score), sq / mbs)
    loss = loss_score + loss_bbox
    return (loss, loss_score, loss_bbox)


# speedup vs baseline: 1.5162x; 1.0297x over previous
import functools

import jax
import jax.numpy as jnp
from jax import lax
from jax.experimental import pallas as pl
from jax.experimental.pallas import tpu as pltpu

B, H, W = 32, 320, 320
HW = H * W
ROWS = HW // 128
N = B * HW
HNM_RATIO = 5
K_ELSE = int(N * 0.1)
MASK30 = (1 << 30) - 1

_RA = 800
_RH = 800

def _srl(x, n):
    return lax.shift_right_logical(x, jnp.int32(n))


def _pass_a(ps_ref, gl_ref, gm2_ref, gm3_ref, gm4_ref, gm5_ref, pbb_ref,
            enc_ref, w_ref, stats_ref, hist_ref):
    b = pl.program_id(0)
    i = pl.program_id(1)
    first = jnp.logical_and(b == 0, i == 0)

    s0 = ps_ref[0, 0, :, :]
    s1 = ps_ref[0, 1, :, :]
    g0 = gl_ref[0, 0, :, :]
    g1 = gl_ref[0, 1, :, :]

    m = jnp.maximum(s0, s1)
    e0 = jnp.exp(s0 - m)
    e1 = jnp.exp(s1 - m)
    z = e0 + e1
    p1 = e1 / z
    logz = jnp.log(z)
    logp0 = (s0 - m) - logz
    logp1 = (s1 - m) - logz

    posf = g0 > 0.5
    negf = g1 > 0.5

    bits = pltpu.bitcast(p1, jnp.int32)
    wv = -g1 * logp1
    enc_ref[0, :, :] = bits | jnp.where(negf, jnp.int32(1 << 30), jnp.int32(0))
    w_ref[0, :, :] = wv

    pos_cnt = jnp.sum(jnp.where(posf, 1.0, 0.0))
    neg_cnt = jnp.sum(jnp.where(negf, 1.0, 0.0))
    term0 = jnp.sum(jnp.where(posf, -g0 * logp0, 0.0))

    sq = jnp.float32(0.0)
    ms = jnp.float32(0.0)
    gm_refs = (gm2_ref, gm3_ref, gm4_ref, gm5_ref)
    for c in range(4):
        mm = gm_refs[c][0, 0, :, :]
        d = pbb_ref[0, c, :, :] * mm - gl_ref[0, 2 + c, :, :] * mm
        sq = sq + jnp.sum(d * d)
        ms = ms + jnp.sum(mm)

    e = _RA * 128
    x = _srl(bits.reshape(1, e), 20)
    xneg = jnp.where(jnp.where(negf, 1, 0).reshape(1, e) == 1, x, 2047)
    xh = _srl(x, 5).astype(jnp.int16)
    xnh = _srl(xneg, 5).astype(jnp.int16)
    xl = (x & 31).astype(jnp.int16)
    iota = lax.broadcasted_iota(jnp.int16, (32, 1), 0)
    one = jnp.bfloat16(1.0)
    zero = jnp.bfloat16(0.0)
    oh_all = jnp.where(xh == iota, one, zero)
    oh_neg = jnp.where(xnh == iota, one, zero)
    lhs = jnp.concatenate([oh_all, oh_neg], axis=0)
    oh_lo = jnp.where(xl == iota, one, zero)
    h = lax.dot_general(lhs, oh_lo, (((1,), (1,)), ((), ())),
                        preferred_element_type=jnp.float32)

    row = lax.broadcasted_iota(jnp.int32, (8, 128), 0)
    col = lax.broadcasted_iota(jnp.int32, (8, 128), 1)
    r0 = row == 0
    sv = (jnp.where(r0 & (col == 0), pos_cnt, 0.0)
          + jnp.where(r0 & (col == 1), neg_cnt, 0.0)
          + jnp.where(r0 & (col == 2), term0, 0.0)
          + jnp.where(r0 & (col == 3), sq, 0.0)
          + jnp.where(r0 & (col == 4), ms, 0.0))

    @pl.when(first)
    def _():
        stats_ref[:, :] = sv
        hist_ref[:, :] = h

    @pl.when(jnp.logical_not(first))
    def _():
        stats_ref[:, :] += sv
        hist_ref[:, :] += h


def _pass_hist(shift, params_ref, enc_ref, hist_ref):
    b = pl.program_id(0)
    i = pl.program_id(1)
    first = jnp.logical_and(b == 0, i == 0)

    has_pos = params_ref[0]
    sel_hi = params_ref[1]

    e = _RH * 128
    enc = enc_ref[0, :, :].reshape(1, e)
    neg = _srl(enc, 30)
    pb = enc & MASK30
    xb = jnp.where(has_pos == 1, jnp.where(neg == 1, pb, 0), pb)

    match = _srl(xb, shift + 10) == sel_hi
    x = jnp.where(match, _srl(xb, shift) & 1023, 2047)

    xh = _srl(x, 5).astype(jnp.int16)
    xl = (x & 31).astype(jnp.int16)
    iota = lax.broadcasted_iota(jnp.int16, (32, 1), 0)
    one = jnp.bfloat16(1.0)
    zero = jnp.bfloat16(0.0)
    oh_hi = jnp.where(xh == iota, one, zero)
    oh_lo = jnp.where(xl == iota, one, zero)
    h = lax.dot_general(oh_hi, oh_lo, (((1,), (1,)), ((), ())),
                        preferred_element_type=jnp.float32)

    @pl.when(first)
    def _():
        hist_ref[:, :] = h

    @pl.when(jnp.logical_not(first))
    def _():
        hist_ref[:, :] += h


def _pass_l3(params_ref, enc_ref, w_ref, hist_ref, tstats_ref):
    b = pl.program_id(0)
    i = pl.program_id(1)
    first = jnp.logical_and(b == 0, i == 0)

    has_pos = params_ref[0]
    pref2 = params_ref[1]

    e = _RH * 128
    enc = enc_ref[0, :, :].reshape(1, e)
    wr = w_ref[0, :, :].reshape(1, e)
    neg = _srl(enc, 30)
    pb = enc & MASK30
    xb = jnp.where(has_pos == 1, jnp.where(neg == 1, pb, 0), pb)

    t_below = jnp.sum(jnp.where(xb < pref2 * 1024, wr, 0.0))

    match = _srl(xb, 10) == pref2
    x = jnp.where(match, xb & 1023, 2047)
    xh = _srl(x, 5).astype(jnp.int16)
    xl = (x & 31).astype(jnp.int16)
    iota = lax.broadcasted_iota(jnp.int16, (32, 1), 0)
    one = jnp.bfloat16(1.0)
    zero = jnp.bfloat16(0.0)
    oh_hi = jnp.where(xh == iota, one, zero)
    oh_lo = jnp.where(xl == iota, one, zero)
    wb = wr.astype(jnp.bfloat16)
    lhs = jnp.concatenate([oh_hi, oh_hi * wb], axis=0)
    h = lax.dot_general(lhs, oh_lo, (((1,), (1,)), ((), ())),
                        preferred_element_type=jnp.float32)

    row = lax.broadcasted_iota(jnp.int32, (8, 128), 0)
    col = lax.broadcasted_iota(jnp.int32, (8, 128), 1)
    sv = jnp.where((row == 0) & (col == 0), t_below, 0.0)

    @pl.when(first)
    def _():
        hist_ref[:, :] = h
        tstats_ref[:, :] = sv

    @pl.when(jnp.logical_not(first))
    def _():
        hist_ref[:, :] += h
        tstats_ref[:, :] += sv


def _hist_call(enc, params, shift):
    return pl.pallas_call(
        functools.partial(_pass_hist, shift),
        grid=(B, ROWS // _RH),
        in_specs=[
            pl.BlockSpec(memory_space=pltpu.SMEM),
            pl.BlockSpec((1, _RH, 128), lambda b, i: (b, i, 0)),
        ],
        out_specs=pl.BlockSpec((32, 32), lambda b, i: (0, 0)),
        out_shape=jax.ShapeDtypeStruct((32, 32), jnp.float32),
    )(params, enc)


def kernel(pred_score, pred_bbox, gt_mask, gt_label):
    ps = pred_score.reshape(B, 2, ROWS, 128)
    pbb = pred_bbox.reshape(B, 4, ROWS, 128)
    gm = gt_mask.reshape(B, 6, ROWS, 128)
    gl = gt_label.reshape(B, 6, ROWS, 128)

    enc, w, stats, hist1 = pl.pallas_call(
        _pass_a,
        grid=(B, ROWS // _RA),
        in_specs=[
            pl.BlockSpec((1, 2, _RA, 128), lambda b, i: (b, 0, i, 0)),
            pl.BlockSpec((1, 6, _RA, 128), lambda b, i: (b, 0, i, 0)),
            pl.BlockSpec((1, 1, _RA, 128), lambda b, i: (b, 2, i, 0)),
            pl.BlockSpec((1, 1, _RA, 128), lambda b, i: (b, 3, i, 0)),
            pl.BlockSpec((1, 1, _RA, 128), lambda b, i: (b, 4, i, 0)),
            pl.BlockSpec((1, 1, _RA, 128), lambda b, i: (b, 5, i, 0)),
            pl.BlockSpec((1, 4, _RA, 128), lambda b, i: (b, 0, i, 0)),
        ],
        out_specs=[
            pl.BlockSpec((1, _RA, 128), lambda b, i: (b, i, 0)),
            pl.BlockSpec((1, _RA, 128), lambda b, i: (b, i, 0)),
            pl.BlockSpec((8, 128), lambda b, i: (0, 0)),
            pl.BlockSpec((64, 32), lambda b, i: (0, 0)),
        ],
        out_shape=[
            jax.ShapeDtypeStruct((B, ROWS, 128), jnp.int32),
            jax.ShapeDtypeStruct((B, ROWS, 128), jnp.float32),
            jax.ShapeDtypeStruct((8, 128), jnp.float32),
            jax.ShapeDtypeStruct((64, 32), jnp.float32),
        ],
    )(ps, gl, gm, gm, gm, gm, pbb)

    pos_num = stats[0, 0].astype(jnp.int32)
    neg_num = stats[0, 1].astype(jnp.int32)
    term0 = stats[0, 2]
    sq = stats[0, 3]
    mbs = stats[0, 4]
    has_pos = pos_num > 0
    hp = has_pos.astype(jnp.int32)

    nns = jnp.minimum(HNM_RATIO * pos_num, neg_num)
    k_pos = jnp.where(nns >= 1, nns, N)
    k = jnp.where(has_pos, k_pos, K_ELSE)

    cnt_all = hist1[0:32, :].reshape(1024).astype(jnp.int32)
    cnt_neg = hist1[32:64, :].reshape(1024).astype(jnp.int32)
    cnt_posb = cnt_neg.at[0].add(N - neg_num)
    h1 = jnp.where(has_pos, cnt_posb, cnt_all)

    c1 = jnp.cumsum(h1)
    sel1 = jnp.argmax(c1 >= k).astype(jnp.int32)
    e1 = c1[sel1] - h1[sel1]
    k2 = k - e1

    h2 = _hist_call(enc, jnp.stack([hp, sel1]), 10).reshape(1024).astype(jnp.int32)
    c2 = jnp.cumsum(h2)
    sel2 = jnp.argmax(c2 >= k2).astype(jnp.int32)
    e2 = c2[sel2] - h2[sel2]
    k3 = k2 - e2

    h3f, tstats = pl.pallas_call(
        _pass_l3,
        grid=(B, ROWS // _RH),
        in_specs=[
            pl.BlockSpec(memory_space=pltpu.SMEM),
            pl.BlockSpec((1, _RH, 128), lambda b, i: (b, i, 0)),
            pl.BlockSpec((1, _RH, 128), lambda b, i: (b, i, 0)),
        ],
        out_specs=[
            pl.BlockSpec((64, 32), lambda b, i: (0, 0)),
            pl.BlockSpec((8, 128), lambda b, i: (0, 0)),
        ],
        out_shape=[
            jax.ShapeDtypeStruct((64, 32), jnp.float32),
            jax.ShapeDtypeStruct((8, 128), jnp.float32),
        ],
    )(jnp.stack([hp, sel1 * 1024 + sel2]), enc, w)

    h3 = h3f[0:32, :].reshape(1024).astype(jnp.int32)
    w3 = h3f[32:64, :].reshape(1024)
    c3 = jnp.cumsum(h3)
    sel3 = jnp.argmax(c3 >= k3).astype(jnp.int32)
    i3 = c3[sel3]
    i3w = jnp.cumsum(w3)[sel3]

    n_sel = (pos_num + e1 + e2 + i3).astype(jnp.float32)
    term1 = tstats[0, 0] + i3w

    loss_score = (term0 + term1) / n_sel
    loss_bbox = jnp.where(mbs == 0, jnp.zeros_like(loss_score), sq / mbs)
    loss = loss_score + loss_bbox
    return (loss, loss_score, loss_bbox)
